# Initial kernel scaffold; baseline (speedup 1.0000x reference)
import jax
import jax.numpy as jnp
from jax.experimental import pallas as pl


def kernel(x, edge_index, W1, b1, W2, b2, Wf, bf):
    raise NotImplementedError("write your pallas kernel here")


# speedup vs baseline: 27.7002x; 27.7002x over previous
import functools

import jax
import jax.numpy as jnp
from jax import lax
from jax.experimental import pallas as pl
from jax.experimental.pallas import tpu as pltpu
from jax.experimental.pallas import tpu_sc as plsc

NC = 2
NS = 16
L = 16
NW = NC * NS

B = 128


def _zero_vmem_1d(ref, n):
    def body(i, _):
        ref[pl.ds(i * L, L)] = jnp.zeros((L,), jnp.float32)
        return 0
    lax.fori_loop(0, n // L, body, 0)


def _fill_vmem_1d(ref, n, val):
    def body(i, _):
        ref[pl.ds(i * L, L)] = jnp.full((L,), val, jnp.float32)
        return 0
    lax.fori_loop(0, n // L, body, 0)


def _zero_vmem_2d(ref, rows, cols):
    def body(i, _):
        r = i // (cols // L)
        k = i % (cols // L)
        ref[r, pl.ds(k * L, L)] = jnp.zeros((L,), jnp.float32)
        return 0
    lax.fori_loop(0, rows * (cols // L), body, 0)


def _make_deg_kernel(NP, KC):
    mesh = plsc.VectorSubcoreMesh(core_axis_name="c", subcore_axis_name="s")
    rows_per_tile = NP // NS

    @functools.partial(
        pl.kernel,
        out_type=jax.ShapeDtypeStruct((NC, NP), jnp.float32),
        mesh=mesh,
        scratch_types=[
            pltpu.VMEM((KC, B), jnp.int32),
            pltpu.VMEM((B,), jnp.float32),
            pltpu.VMEM((rows_per_tile,), jnp.float32),
            pltpu.VMEM_SHARED((NP,), jnp.float32),
        ],
    )
    def deg_kernel(dst3_hbm, deg_out, idx_v, ones_v, zeros_v, deg_sp):
        c = lax.axis_index("c")
        s = lax.axis_index("s")
        wid = c * NS + s

        _fill_vmem_1d(ones_v, B, 1.0)
        _zero_vmem_1d(zeros_v, rows_per_tile)
        pltpu.sync_copy(zeros_v, deg_sp.at[pl.ds(s * rows_per_tile, rows_per_tile)])
        plsc.subcore_barrier()

        pltpu.sync_copy(dst3_hbm.at[wid], idx_v)

        def body(j, _):
            pltpu.sync_copy(ones_v, deg_sp.at[idx_v.at[j]], add=True)
            return 0
        lax.fori_loop(0, KC, body, 0)

        plsc.subcore_barrier()
        pltpu.sync_copy(
            deg_sp.at[pl.ds(s * rows_per_tile, rows_per_tile)],
            deg_out.at[c, pl.ds(s * rows_per_tile, rows_per_tile)],
        )

    return deg_kernel


def _mm_scale_body(x_ref, w1_ref, deg_ref, dinv_ref, hs_ref):
    deg2 = deg_ref[...]
    deg = (deg2[0] + deg2[1] + 1.0)[:, None]
    dinv = lax.rsqrt(deg)
    dinv_ref[...] = dinv
    h = jnp.dot(x_ref[...], w1_ref[...], preferred_element_type=jnp.float32)
    hs_ref[...] = h * dinv


def _make_mm_scale(NP, D, FP, blk):
    grid = NP // blk
    return pl.pallas_call(
        _mm_scale_body,
        grid=(grid,),
        in_specs=[
            pl.BlockSpec((blk, D), lambda i: (i, 0)),
            pl.BlockSpec((D, FP), lambda i: (0, 0)),
            pl.BlockSpec((NC, blk), lambda i: (0, i)),
        ],
        out_specs=[
            pl.BlockSpec((blk, 1), lambda i: (i, 0)),
            pl.BlockSpec((blk, FP), lambda i: (i, 0)),
        ],
        out_shape=[
            jax.ShapeDtypeStruct((NP, 1), jnp.float32),
            jax.ShapeDtypeStruct((NP, FP), jnp.float32),
        ],
    )


def _make_edge_kernel(NP, FP, KC):
    mesh = plsc.VectorSubcoreMesh(core_axis_name="c", subcore_axis_name="s")
    rows_per_tile = NP // NS

    @functools.partial(
        pl.kernel,
        out_type=[
            jax.ShapeDtypeStruct((NC, NP, FP), jnp.float32),
            jax.ShapeDtypeStruct((NC, NP), jnp.float32),
        ],
        mesh=mesh,
        scratch_types=[
            pltpu.VMEM((KC, B), jnp.int32),
            pltpu.VMEM((KC, B), jnp.int32),
            pltpu.VMEM((B, FP), jnp.float32),
            pltpu.VMEM((B,), jnp.float32),
            pltpu.VMEM((NP,), jnp.float32),
            pltpu.VMEM((L, FP), jnp.float32),
            pltpu.VMEM((NP // NS,), jnp.float32),
            pltpu.VMEM_SHARED((NP, FP), jnp.float32),
            pltpu.VMEM_SHARED((NP,), jnp.float32),
            pltpu.SemaphoreType.DMA,
        ],
        compiler_params=pltpu.CompilerParams(needs_layout_passes=False),
    )
    def edge_kernel(src3_hbm, dst3_hbm, hs_hbm, dinv_hbm,
                    agg_out, r_out,
                    sidx, didx, rows_v, dvals, dinv_t, zrows, zvec,
                    agg_sp, r_sp, sem):
        c = lax.axis_index("c")
        s = lax.axis_index("s")
        wid = c * NS + s

        _zero_vmem_2d(zrows, L, FP)
        nz = rows_per_tile // L
        def zbody(i, _):
            pltpu.sync_copy(
                zrows, agg_sp.at[pl.ds(s * rows_per_tile + i * L, L)])
            return 0
        lax.fori_loop(0, nz, zbody, 0)
        _zero_vmem_1d(zvec, rows_per_tile)
        pltpu.sync_copy(zvec, r_sp.at[pl.ds(s * rows_per_tile, rows_per_tile)])

        pltpu.sync_copy(src3_hbm.at[wid], sidx)
        pltpu.sync_copy(dst3_hbm.at[wid], didx)
        pltpu.sync_copy(dinv_hbm, dinv_t)
        plsc.subcore_barrier()

        def body(j, _):
            pltpu.async_copy(hs_hbm.at[sidx.at[j]], rows_v, sem).wait()
            pltpu.sync_copy(rows_v, agg_sp.at[didx.at[j]], add=True)
            def gbody(t, _):
                iv = didx[j, pl.ds(t * L, L)]
                dvals[pl.ds(t * L, L)] = plsc.load_gather(dinv_t, [iv])
                return 0
            lax.fori_loop(0, B // L, gbody, 0)
            pltpu.sync_copy(dvals, r_sp.at[sidx.at[j]], add=True)
            return 0
        lax.fori_loop(0, KC, body, 0)

        plsc.subcore_barrier()
        pltpu.sync_copy(
            agg_sp.at[pl.ds(s * rows_per_tile, rows_per_tile)],
            agg_out.at[c, pl.ds(s * rows_per_tile, rows_per_tile)],
        )
        pltpu.sync_copy(
            r_sp.at[pl.ds(s * rows_per_tile, rows_per_tile)],
            r_out.at[c, pl.ds(s * rows_per_tile, rows_per_tile)],
        )

    return edge_kernel


def _make_final(NP, N, F1, FP, blk):
    grid = NP // blk

    def body(agg_ref, hs_ref, dinv_ref, r_ref, b1_ref,
             w2_ref, b2_ref, wf_ref, bf_ref, out_ref, acc):
        i = pl.program_id(0)
        agg2 = agg_ref[...]
        agg = agg2[0] + agg2[1]
        dinv = dinv_ref[...]
        r2 = r_ref[...]
        r = (r2[0] + r2[1])[:, None]
        h1 = jnp.maximum(dinv * (agg + hs_ref[...]) + b1_ref[...], 0.0)
        cvec = dinv * (r + dinv)
        row = i * blk + lax.broadcasted_iota(jnp.int32, (blk, 1), 0)
        cvec = jnp.where(row < N, cvec, 0.0)
        part = jnp.sum(cvec * h1, axis=0, keepdims=True)

        @pl.when(i == 0)
        def _():
            acc[...] = jnp.zeros_like(acc)
        acc[...] += part

        @pl.when(i == grid - 1)
        def _():
            s = acc[...] * (1.0 / N)
            t1 = jnp.dot(s, w2_ref[...], preferred_element_type=jnp.float32)
            t1 = t1 + b2_ref[...]
            o = jnp.dot(t1, wf_ref[...], preferred_element_type=jnp.float32)
            out_ref[...] = o + bf_ref[...]

    return pl.pallas_call(
        body,
        grid=(grid,),
        in_specs=[
            pl.BlockSpec((NC, blk, FP), lambda i: (0, i, 0)),
            pl.BlockSpec((blk, FP), lambda i: (i, 0)),
            pl.BlockSpec((blk, 1), lambda i: (i, 0)),
            pl.BlockSpec((NC, blk), lambda i: (0, i)),
            pl.BlockSpec((1, FP), lambda i: (0, 0)),
            pl.BlockSpec((FP, 128), lambda i: (0, 0)),
            pl.BlockSpec((1, 128), lambda i: (0, 0)),
            pl.BlockSpec((128, 128), lambda i: (0, 0)),
            pl.BlockSpec((1, 128), lambda i: (0, 0)),
        ],
        out_specs=pl.BlockSpec((1, 128), lambda i: (0, 0)),
        out_shape=jax.ShapeDtypeStruct((1, 128), jnp.float32),
        scratch_shapes=[pltpu.VMEM((1, FP), jnp.float32)],
    )


def kernel(x, edge_index, W1, b1, W2, b2, Wf, bf):
    N, D = x.shape
    E = edge_index.shape[1]
    F1 = W1.shape[1]
    FP = 128

    blk = 1024
    NP = ((N + blk) // blk) * blk
    npad_rows = NP - N
    EP = ((E + NW * B - 1) // (NW * B)) * (NW * B)
    KC = EP // (NW * B)

    src = edge_index[0]
    dst = edge_index[1]
    pad_idx = (N + (jnp.arange(EP - E, dtype=jnp.int32) % npad_rows)).astype(jnp.int32)
    src3 = jnp.concatenate([src, pad_idx]).reshape(NW, KC, B)
    dst3 = jnp.concatenate([dst, pad_idx]).reshape(NW, KC, B)
    x_pad = jnp.pad(x, ((0, NP - N), (0, 0)))

    W1p = jnp.pad(W1, ((0, 0), (0, FP - F1)))
    degp = _make_deg_kernel(NP, KC)(dst3)
    dinv2, hs = _make_mm_scale(NP, D, FP, blk)(x_pad, W1p, degp)
    dinv1 = dinv2.reshape(NP)
    aggp, rp = _make_edge_kernel(NP, FP, KC)(src3, dst3, hs, dinv1)

    b1r = jnp.pad(b1, (0, FP - F1)).reshape(1, FP)
    W2p = jnp.pad(W2, ((0, FP - W2.shape[0]), (0, 128 - W2.shape[1])))
    b2p = jnp.pad(b2, (0, 128 - b2.shape[0])).reshape(1, 128)
    Wfp = jnp.pad(Wf, ((0, 128 - Wf.shape[0]), (0, 128 - Wf.shape[1])))
    bfp = jnp.pad(bf, (0, 128 - bf.shape[0])).reshape(1, 128)

    outp = _make_final(NP, N, F1, FP, blk)(
        aggp, hs, dinv2, rp, b1r, W2p, b2p, Wfp, bfp)
    return outp[:, :bf.shape[0]]


# speedup vs baseline: 36.7504x; 1.3267x over previous
import functools

import jax
import jax.numpy as jnp
from jax import lax
from jax.experimental import pallas as pl
from jax.experimental.pallas import tpu as pltpu
from jax.experimental.pallas import tpu_sc as plsc

NC = 2
NS = 16
L = 16
NW = NC * NS

B = 128


def _zero_vmem_1d(ref, n):
    def body(i, _):
        ref[pl.ds(i * L, L)] = jnp.zeros((L,), jnp.float32)
        return 0
    lax.fori_loop(0, n // L, body, 0)


def _fill_vmem_1d(ref, n, val):
    def body(i, _):
        ref[pl.ds(i * L, L)] = jnp.full((L,), val, jnp.float32)
        return 0
    lax.fori_loop(0, n // L, body, 0)


def _zero_vmem_2d(ref, rows, cols):
    def body(i, _):
        r = i // (cols // L)
        k = i % (cols // L)
        ref[r, pl.ds(k * L, L)] = jnp.zeros((L,), jnp.float32)
        return 0
    lax.fori_loop(0, rows * (cols // L), body, 0)


def _make_deg_kernel(NP, KC):
    mesh = plsc.VectorSubcoreMesh(core_axis_name="c", subcore_axis_name="s")
    rows_per_tile = NP // NS

    @functools.partial(
        pl.kernel,
        out_type=jax.ShapeDtypeStruct((NC, NP), jnp.float32),
        mesh=mesh,
        scratch_types=[
            pltpu.VMEM((KC, B), jnp.int32),
            pltpu.VMEM((B,), jnp.float32),
            pltpu.VMEM((rows_per_tile,), jnp.float32),
            pltpu.VMEM_SHARED((NP,), jnp.float32),
        ],
    )
    def deg_kernel(dst3_hbm, deg_out, idx_v, ones_v, zeros_v, deg_sp):
        c = lax.axis_index("c")
        s = lax.axis_index("s")
        wid = c * NS + s

        _fill_vmem_1d(ones_v, B, 1.0)
        _zero_vmem_1d(zeros_v, rows_per_tile)
        pltpu.sync_copy(zeros_v, deg_sp.at[pl.ds(s * rows_per_tile, rows_per_tile)])
        plsc.subcore_barrier()

        pltpu.sync_copy(dst3_hbm.at[wid], idx_v)

        def body(j, _):
            pltpu.sync_copy(ones_v, deg_sp.at[idx_v.at[j]], add=True)
            return 0
        lax.fori_loop(0, KC, body, 0)

        plsc.subcore_barrier()
        pltpu.sync_copy(
            deg_sp.at[pl.ds(s * rows_per_tile, rows_per_tile)],
            deg_out.at[c, pl.ds(s * rows_per_tile, rows_per_tile)],
        )

    return deg_kernel


def _mm_scale_body(x_ref, w1_ref, deg_ref, dinv_ref, hs_ref):
    deg2 = deg_ref[...]
    deg = (deg2[0] + deg2[1] + 1.0)[:, None]
    dinv = lax.rsqrt(deg)
    dinv_ref[...] = dinv
    h = jnp.dot(x_ref[...], w1_ref[...], preferred_element_type=jnp.float32)
    hs_ref[...] = h * dinv


def _make_mm_scale(NP, D, FP, blk):
    grid = NP // blk
    return pl.pallas_call(
        _mm_scale_body,
        grid=(grid,),
        in_specs=[
            pl.BlockSpec((blk, D), lambda i: (i, 0)),
            pl.BlockSpec((D, FP), lambda i: (0, 0)),
            pl.BlockSpec((NC, blk), lambda i: (0, i)),
        ],
        out_specs=[
            pl.BlockSpec((blk, 1), lambda i: (i, 0)),
            pl.BlockSpec((blk, FP), lambda i: (i, 0)),
        ],
        out_shape=[
            jax.ShapeDtypeStruct((NP, 1), jnp.float32),
            jax.ShapeDtypeStruct((NP, FP), jnp.float32),
        ],
    )


def _make_edge_kernel(NP, FP, KC):
    mesh = plsc.VectorSubcoreMesh(core_axis_name="c", subcore_axis_name="s")
    rows_per_tile = NP // NS

    @functools.partial(
        pl.kernel,
        out_type=[
            jax.ShapeDtypeStruct((NC, NP, FP), jnp.float32),
            jax.ShapeDtypeStruct((NC, NP), jnp.float32),
        ],
        mesh=mesh,
        scratch_types=[
            pltpu.VMEM((KC, B), jnp.int32),
            pltpu.VMEM((KC, B), jnp.int32),
            pltpu.VMEM((2, B, FP), jnp.float32),
            pltpu.VMEM((2, B), jnp.float32),
            pltpu.VMEM((L, FP), jnp.float32),
            pltpu.VMEM((NP // NS,), jnp.float32),
            pltpu.VMEM_SHARED((NP, FP), jnp.float32),
            pltpu.VMEM_SHARED((NP,), jnp.float32),
            pltpu.VMEM_SHARED((NP,), jnp.float32),
            pltpu.SemaphoreType.DMA,
            pltpu.SemaphoreType.DMA,
            pltpu.SemaphoreType.DMA,
            pltpu.SemaphoreType.DMA,
        ],
        compiler_params=pltpu.CompilerParams(needs_layout_passes=False),
    )
    def edge_kernel(src3_hbm, dst3_hbm, hs_hbm, dinv_hbm,
                    agg_out, r_out,
                    sidx, didx, rows_v, dvals, zrows, zvec,
                    agg_sp, r_sp, dinv_sp,
                    sem0, sem1, dsem0, dsem1, ):
        c = lax.axis_index("c")
        s = lax.axis_index("s")
        wid = c * NS + s
        gsem = (sem0, sem1)
        dsem = (dsem0, dsem1)

        _zero_vmem_2d(zrows, L, FP)
        nz = rows_per_tile // L
        def zbody(i, _):
            pltpu.sync_copy(
                zrows, agg_sp.at[pl.ds(s * rows_per_tile + i * L, L)])
            return 0
        lax.fori_loop(0, nz, zbody, 0)
        _zero_vmem_1d(zvec, rows_per_tile)
        pltpu.sync_copy(zvec, r_sp.at[pl.ds(s * rows_per_tile, rows_per_tile)])

        pltpu.sync_copy(src3_hbm.at[wid], sidx)
        pltpu.sync_copy(dst3_hbm.at[wid], didx)
        @pl.when(s == 0)
        def _():
            pltpu.sync_copy(dinv_hbm, dinv_sp)
        plsc.subcore_barrier()

        def start(j, b):
            pltpu.async_copy(hs_hbm.at[sidx.at[j]], rows_v.at[b], gsem[b])
            pltpu.async_copy(dinv_sp.at[didx.at[j]], dvals.at[b], dsem[b])

        def finish(j, b):
            pltpu.make_async_copy(
                hs_hbm.at[sidx.at[j]], rows_v.at[b], gsem[b]).wait()
            pltpu.sync_copy(rows_v.at[b], agg_sp.at[didx.at[j]], add=True)
            pltpu.make_async_copy(
                dinv_sp.at[didx.at[j]], dvals.at[b], dsem[b]).wait()
            pltpu.sync_copy(dvals.at[b], r_sp.at[sidx.at[j]], add=True)

        start(0, 0)

        def body(jj, _):
            for b in range(2):
                j = jj * 2 + b
                @pl.when(j + 1 < KC)
                def _():
                    start(j + 1, 1 - b)
                finish(j, b)
            return 0
        lax.fori_loop(0, KC // 2, body, 0)

        plsc.subcore_barrier()
        pltpu.sync_copy(
            agg_sp.at[pl.ds(s * rows_per_tile, rows_per_tile)],
            agg_out.at[c, pl.ds(s * rows_per_tile, rows_per_tile)],
        )
        pltpu.sync_copy(
            r_sp.at[pl.ds(s * rows_per_tile, rows_per_tile)],
            r_out.at[c, pl.ds(s * rows_per_tile, rows_per_tile)],
        )

    return edge_kernel


def _make_final(NP, N, F1, FP, blk):
    grid = NP // blk

    def body(agg_ref, hs_ref, dinv_ref, r_ref, b1_ref,
             w2_ref, b2_ref, wf_ref, bf_ref, out_ref, acc):
        i = pl.program_id(0)
        agg2 = agg_ref[...]
        agg = agg2[0] + agg2[1]
        dinv = dinv_ref[...]
        r2 = r_ref[...]
        r = (r2[0] + r2[1])[:, None]
        h1 = jnp.maximum(dinv * (agg + hs_ref[...]) + b1_ref[...], 0.0)
        cvec = dinv * (r + dinv)
        row = i * blk + lax.broadcasted_iota(jnp.int32, (blk, 1), 0)
        cvec = jnp.where(row < N, cvec, 0.0)
        part = jnp.sum(cvec * h1, axis=0, keepdims=True)

        @pl.when(i == 0)
        def _():
            acc[...] = jnp.zeros_like(acc)
        acc[...] += part

        @pl.when(i == grid - 1)
        def _():
            s = acc[...] * (1.0 / N)
            t1 = jnp.dot(s, w2_ref[...], preferred_element_type=jnp.float32)
            t1 = t1 + b2_ref[...]
            o = jnp.dot(t1, wf_ref[...], preferred_element_type=jnp.float32)
            out_ref[...] = o + bf_ref[...]

    return pl.pallas_call(
        body,
        grid=(grid,),
        in_specs=[
            pl.BlockSpec((NC, blk, FP), lambda i: (0, i, 0)),
            pl.BlockSpec((blk, FP), lambda i: (i, 0)),
            pl.BlockSpec((blk, 1), lambda i: (i, 0)),
            pl.BlockSpec((NC, blk), lambda i: (0, i)),
            pl.BlockSpec((1, FP), lambda i: (0, 0)),
            pl.BlockSpec((FP, 128), lambda i: (0, 0)),
            pl.BlockSpec((1, 128), lambda i: (0, 0)),
            pl.BlockSpec((128, 128), lambda i: (0, 0)),
            pl.BlockSpec((1, 128), lambda i: (0, 0)),
        ],
        out_specs=pl.BlockSpec((1, 128), lambda i: (0, 0)),
        out_shape=jax.ShapeDtypeStruct((1, 128), jnp.float32),
        scratch_shapes=[pltpu.VMEM((1, FP), jnp.float32)],
    )


def kernel(x, edge_index, W1, b1, W2, b2, Wf, bf):
    N, D = x.shape
    E = edge_index.shape[1]
    F1 = W1.shape[1]
    FP = 128

    blk = 1024
    NP = ((N + blk) // blk) * blk
    npad_rows = NP - N
    EP = ((E + NW * B - 1) // (NW * B)) * (NW * B)
    KC = EP // (NW * B)

    src = edge_index[0]
    dst = edge_index[1]
    pad_idx = (N + (jnp.arange(EP - E, dtype=jnp.int32) % npad_rows)).astype(jnp.int32)
    src3 = jnp.concatenate([src, pad_idx]).reshape(NW, KC, B)
    dst3 = jnp.concatenate([dst, pad_idx]).reshape(NW, KC, B)
    x_pad = jnp.pad(x, ((0, NP - N), (0, 0)))

    W1p = jnp.pad(W1, ((0, 0), (0, FP - F1)))
    degp = _make_deg_kernel(NP, KC)(dst3)
    dinv2, hs = _make_mm_scale(NP, D, FP, blk)(x_pad, W1p, degp)
    dinv1 = dinv2.reshape(NP)
    aggp, rp = _make_edge_kernel(NP, FP, KC)(src3, dst3, hs, dinv1)

    b1r = jnp.pad(b1, (0, FP - F1)).reshape(1, FP)
    W2p = jnp.pad(W2, ((0, FP - W2.shape[0]), (0, 128 - W2.shape[1])))
    b2p = jnp.pad(b2, (0, 128 - b2.shape[0])).reshape(1, 128)
    Wfp = jnp.pad(Wf, ((0, 128 - Wf.shape[0]), (0, 128 - Wf.shape[1])))
    bfp = jnp.pad(bf, (0, 128 - bf.shape[0])).reshape(1, 128)

    outp = _make_final(NP, N, F1, FP, blk)(
        aggp, hs, dinv2, rp, b1r, W2p, b2p, Wfp, bfp)
    return outp[:, :bf.shape[0]]
